# baseline (speedup 1.0000x reference)
import functools

import jax
import jax.numpy as jnp
from jax import lax
from jax.experimental import pallas as pl
from jax.experimental.pallas import tpu as pltpu
from jax.experimental.pallas import tpu_sc as plsc

_NC = 2
_NS = 16
_NW = _NC * _NS

_CHUNK = 80


@functools.partial(jax.jit, static_argnames=("b_per_w", "d"))
def _sc_gather(table, idx, *, b_per_w, d):
    n_chunks = b_per_w // _CHUNK
    assert n_chunks % 2 == 0 and n_chunks >= 4
    mesh = plsc.VectorSubcoreMesh(core_axis_name="c", subcore_axis_name="s")

    @functools.partial(
        pl.kernel,
        mesh=mesh,
        out_type=jax.ShapeDtypeStruct((b_per_w * _NW, d), jnp.float32),
        scratch_types=[
            pltpu.VMEM((b_per_w,), jnp.int32),
            pltpu.VMEM((2, _CHUNK, d), jnp.float32),
            pltpu.SemaphoreType.DMA,
            pltpu.SemaphoreType.DMA,
            pltpu.SemaphoreType.DMA,
            pltpu.SemaphoreType.DMA,
        ],
    )
    def k(table_hbm, idx_hbm, out_hbm, idx_v, rows_v, gsem0, gsem1, wsem0, wsem1):
        wid = lax.axis_index("s") * _NC + lax.axis_index("c")
        base = wid * b_per_w
        pltpu.sync_copy(idx_hbm.at[pl.ds(base, b_per_w)], idx_v)

        bufs = (rows_v.at[0], rows_v.at[1])
        gsems = (gsem0, gsem1)
        wsems = (wsem0, wsem1)

        def start_gather(i, b):
            pltpu.async_copy(
                table_hbm.at[idx_v.at[pl.ds(i * _CHUNK, _CHUNK)]], bufs[b], gsems[b]
            )

        def wait_gather(b):
            pltpu.make_async_copy(
                table_hbm.at[idx_v.at[pl.ds(0, _CHUNK)]], bufs[b], gsems[b]
            ).wait()

        def start_write(i, b):
            pltpu.async_copy(
                bufs[b], out_hbm.at[pl.ds(base + i * _CHUNK, _CHUNK)], wsems[b]
            )

        def wait_write(b):
            pltpu.make_async_copy(
                bufs[b], out_hbm.at[pl.ds(base, _CHUNK)], wsems[b]
            ).wait()

        start_gather(0, 0)
        start_gather(1, 1)

        @pl.loop(0, n_chunks // 2 - 1)
        def _(j):
            i0 = 2 * j
            wait_gather(0)
            start_write(i0, 0)
            wait_gather(1)
            start_write(i0 + 1, 1)
            wait_write(0)
            start_gather(i0 + 2, 0)
            wait_write(1)
            start_gather(i0 + 3, 1)

        wait_gather(0)
        start_write(n_chunks - 2, 0)
        wait_gather(1)
        start_write(n_chunks - 1, 1)
        wait_write(0)
        wait_write(1)

    return k(table, idx)


def kernel(inputs, kernel):
    table = kernel
    ids = inputs.reshape(-1).astype(jnp.int32)
    b = ids.shape[0]
    d = table.shape[1]
    assert b % (_NW * _CHUNK) == 0
    out = _sc_gather(table, ids, b_per_w=b // _NW, d=d)
    return out.reshape(inputs.shape + (d,))
